# baseline (speedup 1.0000x reference)
import jax
import jax.numpy as jnp
from jax.experimental import pallas as pl
from jax.experimental.pallas import tpu as pltpu

N = 1024
NFEAT = 128
NHID = 16
NOUT = 128
NHEADS = 8
ALPHA = 0.2
NCORE = 4
RB = N // NCORE


def _weights(adj_blk, b, db, r):
    return adj_blk * jnp.minimum(b, r * db)


def _elu(v):
    return jnp.where(v > 0, v, jnp.exp(v) - 1.0)


def _layer1_kernel(x_ref, xblk_ref, adj_ref, wall_ref, a1_ref, a2_ref,
                   x2_ref):
    adj_blk = adj_ref[...]
    h_all = jnp.dot(x_ref[...], wall_ref[...],
                    preferred_element_type=jnp.float32)
    h_blk = jnp.dot(xblk_ref[...], wall_ref[...],
                    preferred_element_type=jnp.float32)
    ones_col = jnp.ones((N, 1), dtype=jnp.float32)
    outs = []
    for hd in range(NHEADS):
        h_i = h_all[:, hd * NHID:(hd + 1) * NHID]
        h_i_blk = h_blk[:, hd * NHID:(hd + 1) * NHID]
        a1 = a1_ref[hd:hd + 1, :]
        a2 = a2_ref[hd:hd + 1, :]
        p = jax.lax.dot_general(h_i_blk, a1, (((1,), (1,)), ((), ())),
                                preferred_element_type=jnp.float32)
        q = jax.lax.dot_general(a2, h_i, (((1,), (1,)), ((), ())),
                                preferred_element_type=jnp.float32)
        b = jnp.exp(-q)
        db = jnp.exp(-ALPHA * q)
        r = jnp.exp((1.0 - ALPHA) * p)
        e = _weights(adj_blk, b, db, r)
        h_aug = jnp.concatenate([h_i, ones_col], axis=1)
        nd = jnp.dot(e, h_aug, preferred_element_type=jnp.float32)
        hp = nd[:, :NHID] * (1.0 / nd[:, NHID:NHID + 1])
        outs.append(_elu(hp))
    x2_ref[...] = jnp.concatenate(outs, axis=1)


def _layer2_kernel(x_ref, x2_ref, x2blk_ref, adj_ref, wout_ref, ao_ref,
                   out_ref):
    adj_blk = adj_ref[...]
    h2 = jnp.dot(x2_ref[...], wout_ref[...],
                 preferred_element_type=jnp.float32)
    h2_blk = jnp.dot(x2blk_ref[...], wout_ref[...],
                     preferred_element_type=jnp.float32)
    p2 = jax.lax.dot_general(h2_blk, ao_ref[:, :NOUT], (((1,), (1,)), ((), ())),
                             preferred_element_type=jnp.float32)
    q2 = jax.lax.dot_general(ao_ref[:, NOUT:], h2, (((1,), (1,)), ((), ())),
                             preferred_element_type=jnp.float32)
    b = jnp.exp(-q2)
    db = jnp.exp(-ALPHA * q2)
    r = jnp.exp((1.0 - ALPHA) * p2)
    e2 = _weights(adj_blk, b, db, r)
    ones_col = jnp.ones((N, 1), dtype=jnp.float32)
    h2_aug = jnp.concatenate([h2, ones_col], axis=1)
    nd = jnp.dot(e2, h2_aug, preferred_element_type=jnp.float32)
    denom = nd[:, NOUT:NOUT + 1]
    h_out = nd[:, :NOUT] * (1.0 / denom)
    h_out = jnp.where(denom == 0.0, x_ref[...], h_out)
    out_ref[...] = _elu(h_out)


def kernel(x, adj, W_heads, a_heads, W_out, a_out):
    w_all = jnp.transpose(W_heads, (1, 0, 2)).reshape(NFEAT, NHEADS * NHID)
    a1_all = a_heads[:, 0, :NHID]
    a2_all = a_heads[:, 0, NHID:]
    mask = adj
    full = lambda *shape: pl.BlockSpec(shape, lambda i: tuple(0 for _ in shape))
    x2 = pl.pallas_call(
        _layer1_kernel,
        grid=(NCORE,),
        in_specs=[
            full(N, NFEAT),
            pl.BlockSpec((RB, NFEAT), lambda i: (i, 0)),
            pl.BlockSpec((RB, N), lambda i: (i, 0)),
            full(NFEAT, NHEADS * NHID),
            full(NHEADS, NHID),
            full(NHEADS, NHID),
        ],
        out_specs=pl.BlockSpec((RB, NHEADS * NHID), lambda i: (i, 0)),
        out_shape=jax.ShapeDtypeStruct((N, NHEADS * NHID), jnp.float32),
        compiler_params=pltpu.CompilerParams(
            dimension_semantics=("parallel",)),
    )(x, x, mask, w_all, a1_all, a2_all)
    return pl.pallas_call(
        _layer2_kernel,
        grid=(NCORE,),
        in_specs=[
            pl.BlockSpec((RB, NFEAT), lambda i: (i, 0)),
            full(N, NHEADS * NHID),
            pl.BlockSpec((RB, NHEADS * NHID), lambda i: (i, 0)),
            pl.BlockSpec((RB, N), lambda i: (i, 0)),
            full(NHEADS * NHID, NOUT),
            full(1, 2 * NOUT),
        ],
        out_specs=pl.BlockSpec((RB, NOUT), lambda i: (i, 0)),
        out_shape=jax.ShapeDtypeStruct((N, NOUT), jnp.float32),
        compiler_params=pltpu.CompilerParams(
            dimension_semantics=("parallel",)),
    )(x, x2, x2, mask, W_out, a_out)


# speedup vs baseline: 1.1748x; 1.1748x over previous
import jax
import jax.numpy as jnp
from jax.experimental import pallas as pl
from jax.experimental.pallas import tpu as pltpu

N = 1024
NFEAT = 128
NHID = 16
NOUT = 128
NHEADS = 8
ALPHA = 0.2
NCORE = 2
RB = N // NCORE


def _weights(adj_blk, b, db, r):
    return adj_blk * jnp.minimum(b, r * db)


def _elu(v):
    return jnp.where(v > 0, v, jnp.exp(v) - 1.0)


def _layer1_kernel(x_ref, xblk_ref, adj_ref, wall_ref, a1_ref, a2_ref,
                   x2_ref):
    adj_blk = adj_ref[...]
    h_all = jnp.dot(x_ref[...], wall_ref[...],
                    preferred_element_type=jnp.float32)
    h_blk = jnp.dot(xblk_ref[...], wall_ref[...],
                    preferred_element_type=jnp.float32)
    ones_col = jnp.ones((N, 1), dtype=jnp.float32)
    outs = []
    for hd in range(NHEADS):
        h_i = h_all[:, hd * NHID:(hd + 1) * NHID]
        h_i_blk = h_blk[:, hd * NHID:(hd + 1) * NHID]
        a1 = a1_ref[hd:hd + 1, :]
        a2 = a2_ref[hd:hd + 1, :]
        p = jax.lax.dot_general(h_i_blk, a1, (((1,), (1,)), ((), ())),
                                preferred_element_type=jnp.float32)
        q = jax.lax.dot_general(a2, h_i, (((1,), (1,)), ((), ())),
                                preferred_element_type=jnp.float32)
        b = jnp.exp(-q)
        db = jnp.exp(-ALPHA * q)
        r = jnp.exp((1.0 - ALPHA) * p)
        e = _weights(adj_blk, b, db, r)
        h_aug = jnp.concatenate([h_i, ones_col], axis=1)
        nd = jnp.dot(e, h_aug, preferred_element_type=jnp.float32)
        hp = nd[:, :NHID] * (1.0 / nd[:, NHID:NHID + 1])
        outs.append(_elu(hp))
    x2_ref[...] = jnp.concatenate(outs, axis=1)


def _layer2_kernel(x_ref, x2_ref, x2blk_ref, adj_ref, wout_ref, ao_ref,
                   out_ref):
    adj_blk = adj_ref[...]
    h2 = jnp.dot(x2_ref[...], wout_ref[...],
                 preferred_element_type=jnp.float32)
    h2_blk = jnp.dot(x2blk_ref[...], wout_ref[...],
                     preferred_element_type=jnp.float32)
    p2 = jax.lax.dot_general(h2_blk, ao_ref[:, :NOUT], (((1,), (1,)), ((), ())),
                             preferred_element_type=jnp.float32)
    q2 = jax.lax.dot_general(ao_ref[:, NOUT:], h2, (((1,), (1,)), ((), ())),
                             preferred_element_type=jnp.float32)
    b = jnp.exp(-q2)
    db = jnp.exp(-ALPHA * q2)
    r = jnp.exp((1.0 - ALPHA) * p2)
    e2 = _weights(adj_blk, b, db, r)
    ones_col = jnp.ones((N, 1), dtype=jnp.float32)
    h2_aug = jnp.concatenate([h2, ones_col], axis=1)
    nd = jnp.dot(e2, h2_aug, preferred_element_type=jnp.float32)
    denom = nd[:, NOUT:NOUT + 1]
    h_out = nd[:, :NOUT] * (1.0 / denom)
    h_out = jnp.where(denom == 0.0, x_ref[...], h_out)
    out_ref[...] = _elu(h_out)


def kernel(x, adj, W_heads, a_heads, W_out, a_out):
    w_all = jnp.transpose(W_heads, (1, 0, 2)).reshape(NFEAT, NHEADS * NHID)
    a1_all = a_heads[:, 0, :NHID]
    a2_all = a_heads[:, 0, NHID:]
    mask = adj
    full = lambda *shape: pl.BlockSpec(shape, lambda i: tuple(0 for _ in shape))
    x2 = pl.pallas_call(
        _layer1_kernel,
        grid=(NCORE,),
        in_specs=[
            full(N, NFEAT),
            pl.BlockSpec((RB, NFEAT), lambda i: (i, 0)),
            pl.BlockSpec((RB, N), lambda i: (i, 0)),
            full(NFEAT, NHEADS * NHID),
            full(NHEADS, NHID),
            full(NHEADS, NHID),
        ],
        out_specs=pl.BlockSpec((RB, NHEADS * NHID), lambda i: (i, 0)),
        out_shape=jax.ShapeDtypeStruct((N, NHEADS * NHID), jnp.float32),
        compiler_params=pltpu.CompilerParams(
            dimension_semantics=("parallel",)),
    )(x, x, mask, w_all, a1_all, a2_all)
    return pl.pallas_call(
        _layer2_kernel,
        grid=(NCORE,),
        in_specs=[
            pl.BlockSpec((RB, NFEAT), lambda i: (i, 0)),
            full(N, NHEADS * NHID),
            pl.BlockSpec((RB, NHEADS * NHID), lambda i: (i, 0)),
            pl.BlockSpec((RB, N), lambda i: (i, 0)),
            full(NHEADS * NHID, NOUT),
            full(1, 2 * NOUT),
        ],
        out_specs=pl.BlockSpec((RB, NOUT), lambda i: (i, 0)),
        out_shape=jax.ShapeDtypeStruct((N, NOUT), jnp.float32),
        compiler_params=pltpu.CompilerParams(
            dimension_semantics=("parallel",)),
    )(x, x2, x2, mask, W_out, a_out)
